# baseline (speedup 1.0000x reference)
import jax
import jax.numpy as jnp
from jax import lax
from jax.experimental import pallas as pl
from jax.experimental.pallas import tpu as pltpu
from jax.experimental.pallas import tpu_sc as plsc

VOCAB = 1000000
EMB_DIM = 32
BATCH = 16384
FIELDS = 26

NC = 2
NS = 16
L = 16
NW = NC * NS

TOTAL = BATCH * FIELDS
PER_W = TOTAL // NW
CHUNK = 128
NCHUNK = PER_W // CHUNK
NBUF = 4


def _body(qw_hbm, idx_hbm, out_hbm, idx_v,
          rows0, rows1, rows2, rows3,
          gs0, gs1, gs2, gs3, ss0, ss1, ss2, ss3):
    wid = lax.axis_index("s") * NC + lax.axis_index("c")

    pltpu.sync_copy(idx_hbm.at[wid], idx_v)

    rows = (rows0, rows1, rows2, rows3)
    gsem = (gs0, gs1, gs2, gs3)
    ssem = (ss0, ss1, ss2, ss3)

    rbase = wid * PER_W

    def fire(c, b):
        pltpu.async_copy(qw_hbm.at[idx_v.at[c]], rows[b], gsem[b])

    def gwait(c, b):
        pltpu.make_async_copy(qw_hbm.at[idx_v.at[c]], rows[b], gsem[b]).wait()

    def sfire(c, b):
        pltpu.async_copy(rows[b],
                         out_hbm.at[pl.ds(rbase + c * CHUNK, CHUNK)], ssem[b])

    def swait(c, b):
        pltpu.make_async_copy(rows[b],
                              out_hbm.at[pl.ds(rbase + c * CHUNK, CHUNK)],
                              ssem[b]).wait()

    for b in range(NBUF):
        fire(b, b)

    def rot_body(q, carry):
        for b in range(NBUF):
            c = NBUF * q + b
            gwait(c, b)
            sfire(c, b)

            @pl.when(c + NBUF < NCHUNK)
            def _():
                swait(c, b)
                fire(c + NBUF, b)
        return carry

    lax.fori_loop(0, NCHUNK // NBUF, rot_body, 0)

    for b in range(NBUF):
        c = NCHUNK - NBUF + b
        swait(c, b)


TVB = 2048


def _transpose_body(qt_ref, o_ref):
    v = qt_ref[...].astype(jnp.float32)
    o_ref[...] = v.T.astype(jnp.int8)


def _dequant_body(g_ref, sb_ref, o_ref):
    v = g_ref[...].astype(jnp.float32)
    y = (v - sb_ref[1]) * sb_ref[0]
    for f in range(FIELDS):
        o_ref[f] = y[:, f * EMB_DIM:(f + 1) * EMB_DIM].T


@jax.jit
def kernel(x, qweight, scale, bias):
    idx = x.reshape(-1).astype(jnp.int32).reshape(NW, NCHUNK, CHUNK)

    qrow = pl.pallas_call(
        _transpose_body,
        grid=((VOCAB + TVB - 1) // TVB,),
        in_specs=[pl.BlockSpec((EMB_DIM, TVB), lambda j: (0, j))],
        out_specs=pl.BlockSpec((TVB, EMB_DIM), lambda j: (j, 0)),
        out_shape=jax.ShapeDtypeStruct((VOCAB, EMB_DIM), jnp.int8),
    )(qweight.T)

    mesh = plsc.VectorSubcoreMesh(core_axis_name="c", subcore_axis_name="s",
                                  num_cores=NC, num_subcores=NS)
    g = pl.kernel(
        _body,
        out_type=jax.ShapeDtypeStruct((TOTAL, EMB_DIM), jnp.int8),
        mesh=mesh,
        compiler_params=pltpu.CompilerParams(needs_layout_passes=False,
                                             use_tc_tiling_on_sc=False),
        scratch_types=[
            pltpu.VMEM((NCHUNK, CHUNK), jnp.int32),
            pltpu.VMEM((CHUNK, EMB_DIM), jnp.int8),
            pltpu.VMEM((CHUNK, EMB_DIM), jnp.int8),
            pltpu.VMEM((CHUNK, EMB_DIM), jnp.int8),
            pltpu.VMEM((CHUNK, EMB_DIM), jnp.int8),
            pltpu.SemaphoreType.DMA,
            pltpu.SemaphoreType.DMA,
            pltpu.SemaphoreType.DMA,
            pltpu.SemaphoreType.DMA,
            pltpu.SemaphoreType.DMA,
            pltpu.SemaphoreType.DMA,
            pltpu.SemaphoreType.DMA,
            pltpu.SemaphoreType.DMA,
        ],
    )(qrow, idx)

    g2 = g.reshape(BATCH, FIELDS * EMB_DIM)
    sb = jnp.stack([scale, bias.astype(jnp.float32)])
    BB = 512
    out_t = pl.pallas_call(
        _dequant_body,
        grid=(BATCH // BB,),
        in_specs=[
            pl.BlockSpec((BB, FIELDS * EMB_DIM), lambda j: (j, 0)),
            pl.BlockSpec(memory_space=pltpu.SMEM),
        ],
        out_specs=pl.BlockSpec((FIELDS, EMB_DIM, BB), lambda j: (0, 0, j)),
        out_shape=jax.ShapeDtypeStruct((FIELDS, EMB_DIM, BATCH), jnp.float32),
    )(g2, sb)
    return out_t.transpose(2, 0, 1)


# speedup vs baseline: 1.3030x; 1.3030x over previous
import jax
import jax.numpy as jnp
from jax import lax
from jax.experimental import pallas as pl
from jax.experimental.pallas import tpu as pltpu
from jax.experimental.pallas import tpu_sc as plsc

VOCAB = 1000000
EMB_DIM = 32
BATCH = 16384
FIELDS = 26

NC = 2
NS = 16
L = 16
NW = NC * NS

TOTAL = BATCH * FIELDS
PER_W = TOTAL // NW
CHUNK = 128
NCHUNK = PER_W // CHUNK
NBUF = 4


def _body(qw_hbm, idx_hbm, out_hbm, idx_v,
          rows0, rows1, rows2, rows3,
          gs0, gs1, gs2, gs3, ss0, ss1, ss2, ss3):
    wid = lax.axis_index("s") * NC + lax.axis_index("c")

    pltpu.sync_copy(idx_hbm.at[wid], idx_v)

    rows = (rows0, rows1, rows2, rows3)
    gsem = (gs0, gs1, gs2, gs3)
    ssem = (ss0, ss1, ss2, ss3)

    rbase = wid * PER_W

    def fire(c, b):
        pltpu.async_copy(qw_hbm.at[idx_v.at[c]], rows[b], gsem[b])

    def gwait(c, b):
        pltpu.make_async_copy(qw_hbm.at[idx_v.at[c]], rows[b], gsem[b]).wait()

    def sfire(c, b):
        pltpu.async_copy(rows[b],
                         out_hbm.at[pl.ds(rbase + c * CHUNK, CHUNK)], ssem[b])

    def swait(c, b):
        pltpu.make_async_copy(rows[b],
                              out_hbm.at[pl.ds(rbase + c * CHUNK, CHUNK)],
                              ssem[b]).wait()

    for b in range(NBUF):
        fire(b, b)

    def rot_body(q, carry):
        for b in range(NBUF):
            c = NBUF * q + b
            gwait(c, b)
            sfire(c, b)

            @pl.when(c + NBUF < NCHUNK)
            def _():
                swait(c, b)
                fire(c + NBUF, b)
        return carry

    lax.fori_loop(0, NCHUNK // NBUF, rot_body, 0)

    for b in range(NBUF):
        c = NCHUNK - NBUF + b
        swait(c, b)


def _dequant_body(g_ref, sb_ref, o_ref):
    v = g_ref[...].astype(jnp.float32)
    y = (v - sb_ref[1]) * sb_ref[0]
    for f in range(FIELDS):
        o_ref[f] = y[:, f * EMB_DIM:(f + 1) * EMB_DIM].T


@jax.jit
def kernel(x, qweight, scale, bias):
    idx = x.reshape(-1).astype(jnp.int32).reshape(NW, NCHUNK, CHUNK)

    mesh = plsc.VectorSubcoreMesh(core_axis_name="c", subcore_axis_name="s",
                                  num_cores=NC, num_subcores=NS)
    g = pl.kernel(
        _body,
        out_type=jax.ShapeDtypeStruct((TOTAL, EMB_DIM), jnp.int8),
        mesh=mesh,
        compiler_params=pltpu.CompilerParams(needs_layout_passes=False,
                                             use_tc_tiling_on_sc=False),
        scratch_types=[
            pltpu.VMEM((NCHUNK, CHUNK), jnp.int32),
            pltpu.VMEM((CHUNK, EMB_DIM), jnp.int8),
            pltpu.VMEM((CHUNK, EMB_DIM), jnp.int8),
            pltpu.VMEM((CHUNK, EMB_DIM), jnp.int8),
            pltpu.VMEM((CHUNK, EMB_DIM), jnp.int8),
            pltpu.SemaphoreType.DMA,
            pltpu.SemaphoreType.DMA,
            pltpu.SemaphoreType.DMA,
            pltpu.SemaphoreType.DMA,
            pltpu.SemaphoreType.DMA,
            pltpu.SemaphoreType.DMA,
            pltpu.SemaphoreType.DMA,
            pltpu.SemaphoreType.DMA,
        ],
    )(qweight, idx)

    g2 = g.reshape(BATCH, FIELDS * EMB_DIM)
    sb = jnp.stack([scale, bias.astype(jnp.float32)])
    BB = 512
    out_t = pl.pallas_call(
        _dequant_body,
        grid=(BATCH // BB,),
        in_specs=[
            pl.BlockSpec((BB, FIELDS * EMB_DIM), lambda j: (j, 0)),
            pl.BlockSpec(memory_space=pltpu.SMEM),
        ],
        out_specs=pl.BlockSpec((FIELDS, EMB_DIM, BB), lambda j: (0, 0, j)),
        out_shape=jax.ShapeDtypeStruct((FIELDS, EMB_DIM, BATCH), jnp.float32),
    )(g2, sb)
    return out_t.transpose(2, 0, 1)
